# baseline (speedup 1.0000x reference)
import jax
import jax.numpy as jnp
from jax.experimental import pallas as pl
from jax.experimental.pallas import tpu as pltpu
from jax.experimental.pallas import tpu_sc as plsc

N_ROWS = 4096
N_COLS = 100000
NEG_INF = float("-inf")
SCALING = 0.5

RPB = 8
SCB = 4096
N_SC_CB = 24
TAIL0 = N_SC_CB * SCB
TAIL_COLS = N_COLS - TAIL0
MW = N_SC_CB * 128

ROWS_B = 256
RSUB = 64
N_RB = N_ROWS // ROWS_B


def _sc_block_body(x_vmem, o_vmem):
    neg = jnp.full((16,), NEG_INF, jnp.float32)
    for r in range(RPB):
        p1 = x_vmem[r, pl.ds(0, 16)]
        p2 = neg
        for k in range(1, SCB // 16):
            xk = x_vmem[r, pl.ds(k * 16, 16)]
            p2 = jnp.maximum(p2, jnp.minimum(p1, xk))
            p1 = jnp.maximum(p1, xk)
        o_vmem[r, pl.ds(0, 16)] = p1
        o_vmem[r, pl.ds(16, 16)] = p2
        for q in range(2, 8):
            o_vmem[r, pl.ds(q * 16, 16)] = neg


def _sc_pass(prediction):
    mesh = plsc.VectorSubcoreMesh(core_axis_name="c", subcore_axis_name="s")

    @pl.kernel(
        out_type=jax.ShapeDtypeStruct((N_ROWS, MW), jnp.float32),
        mesh=mesh,
    )
    def sc_kernel(x_hbm, o_hbm):
        pltpu.emit_pipeline(
            _sc_block_body,
            grid=(N_ROWS // RPB, N_SC_CB),
            in_specs=[pl.BlockSpec((RPB, SCB), lambda i, j: (i, j))],
            out_specs=[pl.BlockSpec((RPB, 128), lambda i, j: (i, j))],
            core_axis_name=("c", "s"),
            dimension_semantics=(pltpu.PARALLEL, pltpu.PARALLEL),
        )(x_hbm, o_hbm)

    return sc_kernel(prediction)


def _merge_body(pairs_ref, tail_ref, o_ref):
    lane = jax.lax.broadcasted_iota(jnp.int32, (1, 128), 1)
    total = jnp.float32(0.0)
    for r in range(0, ROWS_B, RSUB):
        rows = pl.ds(r, RSUB)
        p1 = jnp.full((RSUB, 128), NEG_INF, jnp.float32)
        p2 = jnp.full((RSUB, 128), NEG_INF, jnp.float32)
        for k in range(MW // 128):
            xk = pairs_ref[rows, pl.ds(k * 128, 128)]
            p2 = jnp.maximum(p2, jnp.minimum(p1, xk))
            p1 = jnp.maximum(p1, xk)
        for k in range((TAIL_COLS + 127) // 128):
            xk = tail_ref[rows, pl.ds(k * 128, 128)]
            rem = TAIL_COLS - k * 128
            if rem < 128:
                xk = jnp.where(lane < rem, xk, NEG_INF)
            p2 = jnp.maximum(p2, jnp.minimum(p1, xk))
            p1 = jnp.maximum(p1, xk)
        m1 = jnp.max(p1, axis=1, keepdims=True)
        eq = p1 == m1
        cnt = jnp.sum(eq.astype(jnp.int32), axis=1, keepdims=True)
        runner = jnp.max(jnp.where(eq, NEG_INF, p1), axis=1, keepdims=True)
        second_p1 = jnp.where(cnt > 1, m1, runner)
        m2 = jnp.maximum(second_p1, jnp.max(p2, axis=1, keepdims=True))
        total += jnp.sum(m1 - m2)
    o_ref[...] = total.reshape(1, 1, 1)


def _merge_pass(pairs, prediction):
    return pl.pallas_call(
        _merge_body,
        grid=(N_RB,),
        in_specs=[
            pl.BlockSpec((ROWS_B, MW), lambda i: (i, 0)),
            pl.BlockSpec((ROWS_B, SCB), lambda i: (i, N_SC_CB)),
        ],
        out_specs=pl.BlockSpec((1, 1, 1), lambda i: (i, 0, 0)),
        out_shape=jax.ShapeDtypeStruct((N_RB, 1, 1), jnp.float32),
    )(pairs, prediction)


def kernel(lipschitz, prediction, target):
    del target
    pairs = _sc_pass(prediction)
    sums = _merge_pass(pairs, prediction)
    return (jnp.sum(sums) / N_ROWS) * SCALING / lipschitz


# speedup vs baseline: 1.3216x; 1.3216x over previous
import jax
import jax.numpy as jnp
from jax.experimental import pallas as pl
from jax.experimental.pallas import tpu as pltpu
from jax.experimental.pallas import tpu_sc as plsc

N_ROWS = 4096
N_COLS = 100000
NEG_INF = float("-inf")
SCALING = 0.5

ROWS_B = 256
RSUB = 64

TC_ROWS = 2304
SC_ROWS = N_ROWS - TC_ROWS
N_TC_RB = TC_ROWS // ROWS_B
N_MG_RB = SC_ROWS // ROWS_B

SCOLS = 2048
NBUF = 8
ALIGN_COLS = (N_COLS // 128) * 128
TC_TAIL = N_COLS - ALIGN_COLS
N_CBLK = (ALIGN_COLS + SCOLS - 1) // SCOLS

RPB = 8
SCB = 4096
N_SC_CB = 24
SC_TAIL0 = N_SC_CB * SCB
SC_TAIL = N_COLS - SC_TAIL0
MW = N_SC_CB * 128


def _merge(p1, p2, xk):
    return jnp.maximum(p1, xk), jnp.maximum(p2, jnp.minimum(p1, xk))


def _pairs_to_margin_sum(p1, p2):
    m1 = jnp.max(p1, axis=1, keepdims=True)
    eq = p1 == m1
    cnt = jnp.sum(eq.astype(jnp.int32), axis=1, keepdims=True)
    runner = jnp.max(jnp.where(eq, NEG_INF, p1), axis=1, keepdims=True)
    second_p1 = jnp.where(cnt > 1, m1, runner)
    m2 = jnp.maximum(second_p1, jnp.max(p2, axis=1, keepdims=True))
    return jnp.sum(m1 - m2)


def _tc_blk_w(c):
    return SCOLS if c < N_CBLK - 1 else ALIGN_COLS - (N_CBLK - 1) * SCOLS


def _tc_copy(x_hbm, row0, c, buf_ref, sem):
    w = _tc_blk_w(c)
    dst = buf_ref if w == SCOLS else buf_ref.at[:, pl.ds(0, w)]
    return pltpu.make_async_copy(
        x_hbm.at[pl.ds(row0, ROWS_B), pl.ds(c * SCOLS, w)],
        dst,
        sem,
    )


def _tc_sweep(buf_ref, p1_ref, p2_ref, c):
    w = _tc_blk_w(c)
    for r in range(0, ROWS_B, RSUB):
        rows = pl.ds(r, RSUB)
        p1 = p1_ref[rows, :]
        p2 = p2_ref[rows, :]
        for k in range(w // 128):
            p1, p2 = _merge(p1, p2, buf_ref[rows, pl.ds(k * 128, 128)])
        p1_ref[rows, :] = p1
        p2_ref[rows, :] = p2


def _tc_body(x_hbm, tail_ref, o_ref, *refs):
    bufs = refs[:NBUF]
    sems = refs[NBUF]
    p1_ref, p2_ref = refs[NBUF + 1:]
    i = pl.program_id(0)
    row0 = i * ROWS_B

    p1_ref[...] = jnp.full((ROWS_B, 128), NEG_INF, jnp.float32)
    p2_ref[...] = jnp.full((ROWS_B, 128), NEG_INF, jnp.float32)

    for c in range(min(NBUF, N_CBLK)):
        _tc_copy(x_hbm, row0, c, bufs[c % NBUF], sems.at[c % NBUF]).start()
    for c in range(N_CBLK):
        b = c % NBUF
        _tc_copy(x_hbm, row0, c, bufs[b], sems.at[b]).wait()
        _tc_sweep(bufs[b], p1_ref, p2_ref, c)
        nxt = c + NBUF
        if nxt < N_CBLK:
            _tc_copy(x_hbm, row0, nxt, bufs[b], sems.at[b]).start()

    lane = jax.lax.broadcasted_iota(jnp.int32, (1, 128), 1)
    for r in range(0, ROWS_B, RSUB):
        rows = pl.ds(r, RSUB)
        xt = jnp.where(lane < TC_TAIL, tail_ref[rows, :], NEG_INF)
        p1, p2 = _merge(p1_ref[rows, :], p2_ref[rows, :], xt)
        p1_ref[rows, :] = p1
        p2_ref[rows, :] = p2

    o_ref[...] = _pairs_to_margin_sum(p1_ref[...], p2_ref[...]).reshape(1, 1, 1)


def _tc_pass(prediction):
    return pl.pallas_call(
        _tc_body,
        grid=(N_TC_RB,),
        in_specs=[
            pl.BlockSpec(memory_space=pl.ANY),
            pl.BlockSpec((ROWS_B, 128), lambda i: (i, ALIGN_COLS // 128)),
        ],
        out_specs=pl.BlockSpec((1, 1, 1), lambda i: (i, 0, 0)),
        out_shape=jax.ShapeDtypeStruct((N_TC_RB, 1, 1), jnp.float32),
        scratch_shapes=[pltpu.VMEM((ROWS_B, SCOLS), jnp.float32)] * NBUF
        + [
            pltpu.SemaphoreType.DMA((NBUF,)),
            pltpu.VMEM((ROWS_B, 128), jnp.float32),
            pltpu.VMEM((ROWS_B, 128), jnp.float32),
        ],
    )(prediction, prediction)


def _sc_block_body(x_vmem, o_vmem):
    neg = jnp.full((16,), NEG_INF, jnp.float32)
    for r in range(RPB):
        p1 = x_vmem[r, pl.ds(0, 16)]
        p2 = neg
        for k in range(1, SCB // 16):
            xk = x_vmem[r, pl.ds(k * 16, 16)]
            p2 = jnp.maximum(p2, jnp.minimum(p1, xk))
            p1 = jnp.maximum(p1, xk)
        o_vmem[r, pl.ds(0, 16)] = p1
        o_vmem[r, pl.ds(16, 16)] = p2
        for q in range(2, 8):
            o_vmem[r, pl.ds(q * 16, 16)] = neg


def _sc_pass(prediction):
    mesh = plsc.VectorSubcoreMesh(core_axis_name="c", subcore_axis_name="s")
    row_blk0 = TC_ROWS // RPB

    @pl.kernel(
        out_type=jax.ShapeDtypeStruct((SC_ROWS, MW), jnp.float32),
        mesh=mesh,
    )
    def sc_kernel(x_hbm, o_hbm):
        pltpu.emit_pipeline(
            _sc_block_body,
            grid=(SC_ROWS // RPB, N_SC_CB),
            in_specs=[pl.BlockSpec((RPB, SCB), lambda i, j: (i + row_blk0, j))],
            out_specs=[pl.BlockSpec((RPB, 128), lambda i, j: (i, j))],
            core_axis_name=("c", "s"),
            dimension_semantics=(pltpu.PARALLEL, pltpu.PARALLEL),
        )(x_hbm, o_hbm)

    return sc_kernel(prediction)


def _mg_body(pairs_ref, tail_ref, o_ref):
    lane = jax.lax.broadcasted_iota(jnp.int32, (1, 128), 1)
    total = jnp.float32(0.0)
    for r in range(0, ROWS_B, RSUB):
        rows = pl.ds(r, RSUB)
        p1 = jnp.full((RSUB, 128), NEG_INF, jnp.float32)
        p2 = jnp.full((RSUB, 128), NEG_INF, jnp.float32)
        for k in range(MW // 128):
            p1, p2 = _merge(p1, p2, pairs_ref[rows, pl.ds(k * 128, 128)])
        for k in range((SC_TAIL + 127) // 128):
            xk = tail_ref[rows, pl.ds(k * 128, 128)]
            rem = SC_TAIL - k * 128
            if rem < 128:
                xk = jnp.where(lane < rem, xk, NEG_INF)
            p1, p2 = _merge(p1, p2, xk)
        total += _pairs_to_margin_sum(p1, p2)
    o_ref[...] = total.reshape(1, 1, 1)


def _mg_pass(pairs, prediction):
    mg_row_blk0 = TC_ROWS // ROWS_B
    return pl.pallas_call(
        _mg_body,
        grid=(N_MG_RB,),
        in_specs=[
            pl.BlockSpec((ROWS_B, MW), lambda i: (i, 0)),
            pl.BlockSpec((ROWS_B, SCB), lambda i: (i + mg_row_blk0, N_SC_CB)),
        ],
        out_specs=pl.BlockSpec((1, 1, 1), lambda i: (i, 0, 0)),
        out_shape=jax.ShapeDtypeStruct((N_MG_RB, 1, 1), jnp.float32),
    )(pairs, prediction)


def kernel(lipschitz, prediction, target):
    del target
    tc_sums = _tc_pass(prediction)
    pairs = _sc_pass(prediction)
    mg_sums = _mg_pass(pairs, prediction)
    total = jnp.sum(tc_sums) + jnp.sum(mg_sums)
    return (total / N_ROWS) * SCALING / lipschitz


# speedup vs baseline: 1.3584x; 1.0278x over previous
import jax
import jax.numpy as jnp
from jax.experimental import pallas as pl
from jax.experimental.pallas import tpu as pltpu
from jax.experimental.pallas import tpu_sc as plsc

N_ROWS = 4096
N_COLS = 100000
NEG_INF = float("-inf")
SCALING = 0.5

ROWS_B = 256
RSUB = 64

TC_ROWS = 2816
SC_ROWS = N_ROWS - TC_ROWS
N_TC_RB = TC_ROWS // ROWS_B
N_MG_RB = SC_ROWS // ROWS_B

SCOLS = 2048
NBUF = 8
ALIGN_COLS = (N_COLS // 128) * 128
TC_TAIL = N_COLS - ALIGN_COLS
N_CBLK = (ALIGN_COLS + SCOLS - 1) // SCOLS

RPB = 8
SCB = 4096
N_SC_CB = 24
SC_TAIL0 = N_SC_CB * SCB
SC_TAIL = N_COLS - SC_TAIL0
MW = N_SC_CB * 128


def _merge(p1, p2, xk):
    return jnp.maximum(p1, xk), jnp.maximum(p2, jnp.minimum(p1, xk))


def _pairs_to_margin_sum(p1, p2):
    m1 = jnp.max(p1, axis=1, keepdims=True)
    eq = p1 == m1
    cnt = jnp.sum(eq.astype(jnp.int32), axis=1, keepdims=True)
    runner = jnp.max(jnp.where(eq, NEG_INF, p1), axis=1, keepdims=True)
    second_p1 = jnp.where(cnt > 1, m1, runner)
    m2 = jnp.maximum(second_p1, jnp.max(p2, axis=1, keepdims=True))
    return jnp.sum(m1 - m2)


def _tc_blk_w(c):
    return SCOLS if c < N_CBLK - 1 else ALIGN_COLS - (N_CBLK - 1) * SCOLS


def _tc_copy(x_hbm, row0, c, buf_ref, sem):
    w = _tc_blk_w(c)
    dst = buf_ref if w == SCOLS else buf_ref.at[:, pl.ds(0, w)]
    return pltpu.make_async_copy(
        x_hbm.at[pl.ds(row0, ROWS_B), pl.ds(c * SCOLS, w)],
        dst,
        sem,
    )


def _tc_sweep(buf_ref, p1_ref, p2_ref, c):
    w = _tc_blk_w(c)
    for r in range(0, ROWS_B, RSUB):
        rows = pl.ds(r, RSUB)
        p1 = p1_ref[rows, :]
        p2 = p2_ref[rows, :]
        for k in range(w // 128):
            p1, p2 = _merge(p1, p2, buf_ref[rows, pl.ds(k * 128, 128)])
        p1_ref[rows, :] = p1
        p2_ref[rows, :] = p2


def _tc_body(x_hbm, tail_ref, o_ref, *refs):
    bufs = refs[:NBUF]
    sems = refs[NBUF]
    p1_ref, p2_ref = refs[NBUF + 1:]
    i = pl.program_id(0)
    row0 = i * ROWS_B

    p1_ref[...] = jnp.full((ROWS_B, 128), NEG_INF, jnp.float32)
    p2_ref[...] = jnp.full((ROWS_B, 128), NEG_INF, jnp.float32)

    for c in range(min(NBUF, N_CBLK)):
        _tc_copy(x_hbm, row0, c, bufs[c % NBUF], sems.at[c % NBUF]).start()
    for c in range(N_CBLK):
        b = c % NBUF
        _tc_copy(x_hbm, row0, c, bufs[b], sems.at[b]).wait()
        _tc_sweep(bufs[b], p1_ref, p2_ref, c)
        nxt = c + NBUF
        if nxt < N_CBLK:
            _tc_copy(x_hbm, row0, nxt, bufs[b], sems.at[b]).start()

    lane = jax.lax.broadcasted_iota(jnp.int32, (1, 128), 1)
    for r in range(0, ROWS_B, RSUB):
        rows = pl.ds(r, RSUB)
        xt = jnp.where(lane < TC_TAIL, tail_ref[rows, :], NEG_INF)
        p1, p2 = _merge(p1_ref[rows, :], p2_ref[rows, :], xt)
        p1_ref[rows, :] = p1
        p2_ref[rows, :] = p2

    o_ref[...] = _pairs_to_margin_sum(p1_ref[...], p2_ref[...]).reshape(1, 1, 1)


def _tc_pass(prediction):
    return pl.pallas_call(
        _tc_body,
        grid=(N_TC_RB,),
        in_specs=[
            pl.BlockSpec(memory_space=pl.ANY),
            pl.BlockSpec((ROWS_B, 128), lambda i: (i, ALIGN_COLS // 128)),
        ],
        out_specs=pl.BlockSpec((1, 1, 1), lambda i: (i, 0, 0)),
        out_shape=jax.ShapeDtypeStruct((N_TC_RB, 1, 1), jnp.float32),
        scratch_shapes=[pltpu.VMEM((ROWS_B, SCOLS), jnp.float32)] * NBUF
        + [
            pltpu.SemaphoreType.DMA((NBUF,)),
            pltpu.VMEM((ROWS_B, 128), jnp.float32),
            pltpu.VMEM((ROWS_B, 128), jnp.float32),
        ],
    )(prediction, prediction)


def _sc_block_body(x_vmem, o_vmem):
    neg = jnp.full((16,), NEG_INF, jnp.float32)
    for r in range(RPB):
        p1 = x_vmem[r, pl.ds(0, 16)]
        p2 = neg
        for k in range(1, SCB // 16):
            xk = x_vmem[r, pl.ds(k * 16, 16)]
            p2 = jnp.maximum(p2, jnp.minimum(p1, xk))
            p1 = jnp.maximum(p1, xk)
        o_vmem[r, pl.ds(0, 16)] = p1
        o_vmem[r, pl.ds(16, 16)] = p2
        for q in range(2, 8):
            o_vmem[r, pl.ds(q * 16, 16)] = neg


def _sc_pass(prediction):
    mesh = plsc.VectorSubcoreMesh(core_axis_name="c", subcore_axis_name="s")
    row_blk0 = TC_ROWS // RPB

    @pl.kernel(
        out_type=jax.ShapeDtypeStruct((SC_ROWS, MW), jnp.float32),
        mesh=mesh,
    )
    def sc_kernel(x_hbm, o_hbm):
        pltpu.emit_pipeline(
            _sc_block_body,
            grid=(SC_ROWS // RPB, N_SC_CB),
            in_specs=[pl.BlockSpec((RPB, SCB), lambda i, j: (i + row_blk0, j))],
            out_specs=[pl.BlockSpec((RPB, 128), lambda i, j: (i, j))],
            core_axis_name=("c", "s"),
            dimension_semantics=(pltpu.PARALLEL, pltpu.PARALLEL),
        )(x_hbm, o_hbm)

    return sc_kernel(prediction)


def _mg_body(pairs_ref, tail_ref, o_ref):
    lane = jax.lax.broadcasted_iota(jnp.int32, (1, 128), 1)
    total = jnp.float32(0.0)
    for r in range(0, ROWS_B, RSUB):
        rows = pl.ds(r, RSUB)
        p1 = jnp.full((RSUB, 128), NEG_INF, jnp.float32)
        p2 = jnp.full((RSUB, 128), NEG_INF, jnp.float32)
        for k in range(MW // 128):
            p1, p2 = _merge(p1, p2, pairs_ref[rows, pl.ds(k * 128, 128)])
        for k in range((SC_TAIL + 127) // 128):
            xk = tail_ref[rows, pl.ds(k * 128, 128)]
            rem = SC_TAIL - k * 128
            if rem < 128:
                xk = jnp.where(lane < rem, xk, NEG_INF)
            p1, p2 = _merge(p1, p2, xk)
        total += _pairs_to_margin_sum(p1, p2)
    o_ref[...] = total.reshape(1, 1, 1)


def _mg_pass(pairs, prediction):
    mg_row_blk0 = TC_ROWS // ROWS_B
    return pl.pallas_call(
        _mg_body,
        grid=(N_MG_RB,),
        in_specs=[
            pl.BlockSpec((ROWS_B, MW), lambda i: (i, 0)),
            pl.BlockSpec((ROWS_B, SCB), lambda i: (i + mg_row_blk0, N_SC_CB)),
        ],
        out_specs=pl.BlockSpec((1, 1, 1), lambda i: (i, 0, 0)),
        out_shape=jax.ShapeDtypeStruct((N_MG_RB, 1, 1), jnp.float32),
    )(pairs, prediction)


def kernel(lipschitz, prediction, target):
    del target
    tc_sums = _tc_pass(prediction)
    pairs = _sc_pass(prediction)
    mg_sums = _mg_pass(pairs, prediction)
    total = jnp.sum(tc_sums) + jnp.sum(mg_sums)
    return (total / N_ROWS) * SCALING / lipschitz


# speedup vs baseline: 1.3672x; 1.0065x over previous
import jax
import jax.numpy as jnp
from jax.experimental import pallas as pl
from jax.experimental.pallas import tpu as pltpu
from jax.experimental.pallas import tpu_sc as plsc

N_ROWS = 4096
N_COLS = 100000
NEG_INF = float("-inf")
SCALING = 0.5

ROWS_B = 256
RSUB = 64

TC_ROWS = 3072
SC_ROWS = N_ROWS - TC_ROWS
N_TC_RB = TC_ROWS // ROWS_B
N_MG_RB = SC_ROWS // ROWS_B

SCOLS = 2048
NBUF = 8
ALIGN_COLS = (N_COLS // 128) * 128
TC_TAIL = N_COLS - ALIGN_COLS
N_CBLK = (ALIGN_COLS + SCOLS - 1) // SCOLS

RPB = 8
SCB = 4096
N_SC_CB = 24
SC_TAIL0 = N_SC_CB * SCB
SC_TAIL = N_COLS - SC_TAIL0
MW = N_SC_CB * 128


def _merge(p1, p2, xk):
    return jnp.maximum(p1, xk), jnp.maximum(p2, jnp.minimum(p1, xk))


def _pairs_to_margin_sum(p1, p2):
    m1 = jnp.max(p1, axis=1, keepdims=True)
    eq = p1 == m1
    cnt = jnp.sum(eq.astype(jnp.int32), axis=1, keepdims=True)
    runner = jnp.max(jnp.where(eq, NEG_INF, p1), axis=1, keepdims=True)
    second_p1 = jnp.where(cnt > 1, m1, runner)
    m2 = jnp.maximum(second_p1, jnp.max(p2, axis=1, keepdims=True))
    return jnp.sum(m1 - m2)


def _tc_blk_w(c):
    return SCOLS if c < N_CBLK - 1 else ALIGN_COLS - (N_CBLK - 1) * SCOLS


def _tc_copy(x_hbm, row0, c, buf_ref, sem):
    w = _tc_blk_w(c)
    dst = buf_ref if w == SCOLS else buf_ref.at[:, pl.ds(0, w)]
    return pltpu.make_async_copy(
        x_hbm.at[pl.ds(row0, ROWS_B), pl.ds(c * SCOLS, w)],
        dst,
        sem,
    )


def _tc_sweep(buf_ref, p1_ref, p2_ref, c):
    w = _tc_blk_w(c)
    for r in range(0, ROWS_B, RSUB):
        rows = pl.ds(r, RSUB)
        p1 = p1_ref[rows, :]
        p2 = p2_ref[rows, :]
        for k in range(w // 128):
            p1, p2 = _merge(p1, p2, buf_ref[rows, pl.ds(k * 128, 128)])
        p1_ref[rows, :] = p1
        p2_ref[rows, :] = p2


def _tc_body(x_hbm, tail_ref, o_ref, *refs):
    bufs = refs[:NBUF]
    sems = refs[NBUF]
    p1_ref, p2_ref = refs[NBUF + 1:]
    i = pl.program_id(0)
    row0 = i * ROWS_B

    p1_ref[...] = jnp.full((ROWS_B, 128), NEG_INF, jnp.float32)
    p2_ref[...] = jnp.full((ROWS_B, 128), NEG_INF, jnp.float32)

    for c in range(min(NBUF, N_CBLK)):
        _tc_copy(x_hbm, row0, c, bufs[c % NBUF], sems.at[c % NBUF]).start()
    for c in range(N_CBLK):
        b = c % NBUF
        _tc_copy(x_hbm, row0, c, bufs[b], sems.at[b]).wait()
        _tc_sweep(bufs[b], p1_ref, p2_ref, c)
        nxt = c + NBUF
        if nxt < N_CBLK:
            _tc_copy(x_hbm, row0, nxt, bufs[b], sems.at[b]).start()

    lane = jax.lax.broadcasted_iota(jnp.int32, (1, 128), 1)
    for r in range(0, ROWS_B, RSUB):
        rows = pl.ds(r, RSUB)
        xt = jnp.where(lane < TC_TAIL, tail_ref[rows, :], NEG_INF)
        p1, p2 = _merge(p1_ref[rows, :], p2_ref[rows, :], xt)
        p1_ref[rows, :] = p1
        p2_ref[rows, :] = p2

    o_ref[...] = _pairs_to_margin_sum(p1_ref[...], p2_ref[...]).reshape(1, 1, 1)


def _tc_pass(prediction):
    return pl.pallas_call(
        _tc_body,
        grid=(N_TC_RB,),
        in_specs=[
            pl.BlockSpec(memory_space=pl.ANY),
            pl.BlockSpec((ROWS_B, 128), lambda i: (i, ALIGN_COLS // 128)),
        ],
        out_specs=pl.BlockSpec((1, 1, 1), lambda i: (i, 0, 0)),
        out_shape=jax.ShapeDtypeStruct((N_TC_RB, 1, 1), jnp.float32),
        scratch_shapes=[pltpu.VMEM((ROWS_B, SCOLS), jnp.float32)] * NBUF
        + [
            pltpu.SemaphoreType.DMA((NBUF,)),
            pltpu.VMEM((ROWS_B, 128), jnp.float32),
            pltpu.VMEM((ROWS_B, 128), jnp.float32),
        ],
    )(prediction, prediction)


def _sc_block_body(x_vmem, o_vmem):
    neg = jnp.full((16,), NEG_INF, jnp.float32)
    for r in range(RPB):
        p1 = x_vmem[r, pl.ds(0, 16)]
        p2 = neg
        for k in range(1, SCB // 16):
            xk = x_vmem[r, pl.ds(k * 16, 16)]
            p2 = jnp.maximum(p2, jnp.minimum(p1, xk))
            p1 = jnp.maximum(p1, xk)
        o_vmem[r, pl.ds(0, 16)] = p1
        o_vmem[r, pl.ds(16, 16)] = p2
        for q in range(2, 8):
            o_vmem[r, pl.ds(q * 16, 16)] = neg


def _sc_pass(prediction):
    mesh = plsc.VectorSubcoreMesh(core_axis_name="c", subcore_axis_name="s")
    row_blk0 = TC_ROWS // RPB

    @pl.kernel(
        out_type=jax.ShapeDtypeStruct((SC_ROWS, MW), jnp.float32),
        mesh=mesh,
    )
    def sc_kernel(x_hbm, o_hbm):
        pltpu.emit_pipeline(
            _sc_block_body,
            grid=(SC_ROWS // RPB, N_SC_CB),
            in_specs=[pl.BlockSpec((RPB, SCB), lambda i, j: (i + row_blk0, j))],
            out_specs=[pl.BlockSpec((RPB, 128), lambda i, j: (i, j))],
            core_axis_name=("c", "s"),
            dimension_semantics=(pltpu.PARALLEL, pltpu.PARALLEL),
        )(x_hbm, o_hbm)

    return sc_kernel(prediction)


def _mg_body(pairs_ref, tail_ref, o_ref):
    lane = jax.lax.broadcasted_iota(jnp.int32, (1, 128), 1)
    total = jnp.float32(0.0)
    for r in range(0, ROWS_B, RSUB):
        rows = pl.ds(r, RSUB)
        p1 = jnp.full((RSUB, 128), NEG_INF, jnp.float32)
        p2 = jnp.full((RSUB, 128), NEG_INF, jnp.float32)
        for k in range(MW // 128):
            p1, p2 = _merge(p1, p2, pairs_ref[rows, pl.ds(k * 128, 128)])
        for k in range((SC_TAIL + 127) // 128):
            xk = tail_ref[rows, pl.ds(k * 128, 128)]
            rem = SC_TAIL - k * 128
            if rem < 128:
                xk = jnp.where(lane < rem, xk, NEG_INF)
            p1, p2 = _merge(p1, p2, xk)
        total += _pairs_to_margin_sum(p1, p2)
    o_ref[...] = total.reshape(1, 1, 1)


def _mg_pass(pairs, prediction):
    mg_row_blk0 = TC_ROWS // ROWS_B
    return pl.pallas_call(
        _mg_body,
        grid=(N_MG_RB,),
        in_specs=[
            pl.BlockSpec((ROWS_B, MW), lambda i: (i, 0)),
            pl.BlockSpec((ROWS_B, SCB), lambda i: (i + mg_row_blk0, N_SC_CB)),
        ],
        out_specs=pl.BlockSpec((1, 1, 1), lambda i: (i, 0, 0)),
        out_shape=jax.ShapeDtypeStruct((N_MG_RB, 1, 1), jnp.float32),
    )(pairs, prediction)


def kernel(lipschitz, prediction, target):
    del target
    tc_sums = _tc_pass(prediction)
    pairs = _sc_pass(prediction)
    mg_sums = _mg_pass(pairs, prediction)
    total = jnp.sum(tc_sums) + jnp.sum(mg_sums)
    return (total / N_ROWS) * SCALING / lipschitz
